# baseline (speedup 1.0000x reference)
import functools

import jax
import jax.numpy as jnp
from jax import lax
from jax.experimental import pallas as pl
from jax.experimental.pallas import tpu as pltpu
from jax.experimental.pallas import tpu_sc as plsc

DIM = 128
NUM_MODELS = 64
TEXT_DIM = 1536
BATCH = 4096

NC = 2
NS = 16
L = 16
NW = NC * NS
ROWS_PER_TILE = DIM // NS
B_PER_TILE = BATCH // NW
KCH = TEXT_DIM // L
MBLK = NUM_MODELS // L


def _rsqrt_newton(x):
    i = plsc.bitcast(x, jnp.int32)
    i = 0x5F3759DF - (i >> 1)
    y = plsc.bitcast(i, jnp.float32)
    for _ in range(3):
        y = y * (1.5 - 0.5 * x * y * y)
    return y


def _sc_body(ids_hbm, prompt_hbm, p_hbm, wt_hbm, wcls_hbm, out_hbm,
             wt_v, prompt_v, p_v, wcls_v, part_v, shared, allpart_v,
             table_v, ids_v, out_v, sem_wt, sem_pr, sem_p, sem_wc, sem_id):
    cid = lax.axis_index("c")
    sid = lax.axis_index("s")
    wid = sid * NC + cid

    cp_wt = pltpu.async_copy(
        wt_hbm.at[pl.ds(sid * ROWS_PER_TILE, ROWS_PER_TILE), :], wt_v, sem_wt)
    cp_pr = pltpu.async_copy(prompt_hbm, prompt_v, sem_pr)
    cp_p = pltpu.async_copy(p_hbm, p_v, sem_p)
    cp_wc = pltpu.async_copy(wcls_hbm.at[0], wcls_v, sem_wc)
    cp_id = pltpu.async_copy(
        ids_hbm.at[pl.ds(wid * B_PER_TILE, B_PER_TILE)], ids_v, sem_id)

    cp_pr.wait()
    cp_wt.wait()

    def pe_step(k, accs):
        pch = prompt_v[pl.ds(k * L, L)]
        return tuple(accs[r] + wt_v[r, pl.ds(k * L, L)] * pch
                     for r in range(ROWS_PER_TILE))

    accs = lax.fori_loop(
        0, KCH, pe_step,
        tuple(jnp.zeros((L,), jnp.float32) for _ in range(ROWS_PER_TILE)))

    cp_p.wait()
    cp_wc.wait()

    s_blk = [jnp.zeros((L,), jnp.float32) for _ in range(MBLK)]
    n2_blk = [jnp.zeros((L,), jnp.float32) for _ in range(MBLK)]
    iota = lax.iota(jnp.int32, L)
    for r in range(ROWS_PER_TILE):
        d = sid * ROWS_PER_TILE + r
        dcol = jnp.broadcast_to(d, (L,)).astype(jnp.int32)
        wvec = plsc.load_gather(wcls_v, [dcol])
        w_d = jnp.sum(accs[r])
        for b in range(MBLK):
            g = plsc.load_gather(p_v, [iota + b * L, dcol])
            s_blk[b] = s_blk[b] + g * wvec * w_d
            n2_blk[b] = n2_blk[b] + g * g

    for b in range(MBLK):
        part_v[pl.ds(b * L, L)] = s_blk[b]
        part_v[pl.ds(NUM_MODELS + b * L, L)] = n2_blk[b]

    pltpu.sync_copy(part_v, shared.at[sid])
    plsc.subcore_barrier()
    pltpu.sync_copy(shared, allpart_v)

    def red_step(t, carry):
        return tuple(
            carry[b] + allpart_v[t, pl.ds(b * L, L)] for b in range(2 * MBLK))

    tot = lax.fori_loop(
        0, NS, red_step,
        tuple(jnp.zeros((L,), jnp.float32) for _ in range(2 * MBLK)))

    for b in range(MBLK):
        inv = _rsqrt_newton(jnp.maximum(tot[MBLK + b], 1e-24))
        table_v[pl.ds(b * L, L)] = tot[b] * inv

    cp_id.wait()
    for j in range(B_PER_TILE // L):
        idx = ids_v[pl.ds(j * L, L)]
        out_v[pl.ds(j * L, L)] = plsc.load_gather(table_v, [idx])
    pltpu.sync_copy(out_v, out_hbm.at[pl.ds(wid * B_PER_TILE, B_PER_TILE)])


@jax.jit
def _sc_kernel(model_id, prompt_embed, P, W_text, W_cls):
    mesh = plsc.VectorSubcoreMesh(core_axis_name="c", subcore_axis_name="s",
                                  num_cores=NC, num_subcores=NS)
    return pl.kernel(
        _sc_body,
        out_type=jax.ShapeDtypeStruct((BATCH,), jnp.float32),
        mesh=mesh,
        scratch_types=[
            pltpu.VMEM((ROWS_PER_TILE, TEXT_DIM), jnp.float32),
            pltpu.VMEM((TEXT_DIM,), jnp.float32),
            pltpu.VMEM((NUM_MODELS, DIM), jnp.float32),
            pltpu.VMEM((DIM,), jnp.float32),
            pltpu.VMEM((2 * NUM_MODELS,), jnp.float32),
            pltpu.VMEM_SHARED((NS, 2 * NUM_MODELS), jnp.float32),
            pltpu.VMEM((NS, 2 * NUM_MODELS), jnp.float32),
            pltpu.VMEM((NUM_MODELS,), jnp.float32),
            pltpu.VMEM((B_PER_TILE,), jnp.int32),
            pltpu.VMEM((B_PER_TILE,), jnp.float32),
            pltpu.SemaphoreType.DMA,
            pltpu.SemaphoreType.DMA,
            pltpu.SemaphoreType.DMA,
            pltpu.SemaphoreType.DMA,
            pltpu.SemaphoreType.DMA,
        ],
        compiler_params=pltpu.CompilerParams(needs_layout_passes=False),
    )(model_id, prompt_embed, P, W_text, W_cls)


def kernel(model_id, prompt_embed, P, W_text, W_cls):
    return _sc_kernel(model_id.astype(jnp.int32), prompt_embed, P,
                      W_text, W_cls)


# speedup vs baseline: 1.1127x; 1.1127x over previous
import jax
import jax.numpy as jnp
from jax import lax
from jax.experimental import pallas as pl
from jax.experimental.pallas import tpu as pltpu
from jax.experimental.pallas import tpu_sc as plsc

DIM = 128
NUM_MODELS = 64
TEXT_DIM = 1536
BATCH = 4096

NC = 2
NS = 16
L = 16
NW = NC * NS
B_PER_TILE = BATCH // NW


def _tc_table_body(prompt_ref, p_ref, wt_ref, wcls_ref, table_ref):
    pe = lax.dot_general(
        prompt_ref[...], wt_ref[...],
        dimension_numbers=(((1,), (1,)), ((), ())),
        preferred_element_type=jnp.float32,
    )
    w = pe * wcls_ref[...]
    p = p_ref[...]
    srow = lax.dot_general(
        w, p, dimension_numbers=(((1,), (1,)), ((), ())),
        preferred_element_type=jnp.float32,
    )
    n2row = lax.dot_general(
        jnp.ones((1, DIM), jnp.float32), p * p,
        dimension_numbers=(((1,), (1,)), ((), ())),
        preferred_element_type=jnp.float32,
    )
    table_ref[...] = srow / jnp.maximum(jnp.sqrt(n2row), 1e-12)


def _sc_gather_body(table_hbm, ids_hbm, out_hbm, table_v, ids_v, out_v,
                    sem_t, sem_id):
    cid = lax.axis_index("c")
    sid = lax.axis_index("s")
    wid = sid * NC + cid
    cp_t = pltpu.async_copy(table_hbm.at[0], table_v, sem_t)
    cp_id = pltpu.async_copy(
        ids_hbm.at[pl.ds(wid * B_PER_TILE, B_PER_TILE)], ids_v, sem_id)
    cp_t.wait()
    cp_id.wait()
    for j in range(B_PER_TILE // L):
        idx = ids_v[pl.ds(j * L, L)]
        out_v[pl.ds(j * L, L)] = plsc.load_gather(table_v, [idx])
    pltpu.sync_copy(out_v, out_hbm.at[pl.ds(wid * B_PER_TILE, B_PER_TILE)])


@jax.jit
def _run(model_id, prompt_embed, P, W_text, W_cls):
    table = pl.pallas_call(
        _tc_table_body,
        out_shape=jax.ShapeDtypeStruct((1, NUM_MODELS), jnp.float32),
    )(prompt_embed.reshape(1, TEXT_DIM), P, W_text, W_cls)
    mesh = plsc.VectorSubcoreMesh(core_axis_name="c", subcore_axis_name="s",
                                  num_cores=NC, num_subcores=NS)
    return pl.kernel(
        _sc_gather_body,
        out_type=jax.ShapeDtypeStruct((BATCH,), jnp.float32),
        mesh=mesh,
        scratch_types=[
            pltpu.VMEM((NUM_MODELS,), jnp.float32),
            pltpu.VMEM((B_PER_TILE,), jnp.int32),
            pltpu.VMEM((B_PER_TILE,), jnp.float32),
            pltpu.SemaphoreType.DMA,
            pltpu.SemaphoreType.DMA,
        ],
        compiler_params=pltpu.CompilerParams(needs_layout_passes=False),
    )(table, model_id)


def kernel(model_id, prompt_embed, P, W_text, W_cls):
    return _run(model_id.astype(jnp.int32), prompt_embed, P, W_text, W_cls)


# speedup vs baseline: 6.2013x; 5.5734x over previous
import jax
import jax.numpy as jnp
from jax import lax
from jax.experimental import pallas as pl

DIM = 128
NUM_MODELS = 64
TEXT_DIM = 1536
BATCH = 4096


def _tc_body(ids_ref, prompt_ref, p_ref, wt_ref, wcls_ref, out_ref):
    pe = lax.dot_general(
        prompt_ref[...], wt_ref[...],
        dimension_numbers=(((1,), (1,)), ((), ())),
        preferred_element_type=jnp.float32,
    )
    w = pe * wcls_ref[...]
    p = p_ref[...]
    srow = lax.dot_general(
        w, p, dimension_numbers=(((1,), (1,)), ((), ())),
        preferred_element_type=jnp.float32,
    )
    n2row = lax.dot_general(
        jnp.ones((1, DIM), jnp.float32), p * p,
        dimension_numbers=(((1,), (1,)), ((), ())),
        preferred_element_type=jnp.float32,
    )
    s = srow / jnp.maximum(jnp.sqrt(n2row), 1e-12)
    ids = ids_ref[...]
    iota = lax.broadcasted_iota(jnp.int32, (NUM_MODELS, BATCH), 0)
    onehot = (iota == ids).astype(jnp.float32)
    out_ref[...] = lax.dot_general(
        s, onehot, dimension_numbers=(((1,), (0,)), ((), ())),
        preferred_element_type=jnp.float32,
    )


def kernel(model_id, prompt_embed, P, W_text, W_cls):
    out = pl.pallas_call(
        _tc_body,
        out_shape=jax.ShapeDtypeStruct((1, BATCH), jnp.float32),
    )(model_id.astype(jnp.int32).reshape(1, BATCH),
      prompt_embed.reshape(1, TEXT_DIM), P, W_text, W_cls)
    return out.reshape(BATCH)


# speedup vs baseline: 6.5722x; 1.0598x over previous
import jax
import jax.numpy as jnp
from jax import lax
from jax.experimental import pallas as pl

DIM = 128
NUM_MODELS = 64
TEXT_DIM = 1536
BATCH = 4096
ROWS = BATCH // 128


def _tc_body(ids_ref, prompt_ref, p_ref, wt_ref, wcls_ref, out_ref):
    pe = lax.dot_general(
        prompt_ref[...], wt_ref[...],
        dimension_numbers=(((1,), (1,)), ((), ())),
        preferred_element_type=jnp.float32,
    )
    w = pe * wcls_ref[...]
    p = p_ref[...]
    srow = lax.dot_general(
        w, p, dimension_numbers=(((1,), (1,)), ((), ())),
        preferred_element_type=jnp.float32,
    )
    n2row = lax.dot_general(
        jnp.ones((1, DIM), jnp.float32), p * p,
        dimension_numbers=(((1,), (1,)), ((), ())),
        preferred_element_type=jnp.float32,
    )
    s = srow / jnp.maximum(jnp.sqrt(n2row), 1e-12)
    sb = jnp.broadcast_to(s, (ROWS, NUM_MODELS))
    out_ref[...] = jnp.take_along_axis(sb, ids_ref[...], axis=1)


def kernel(model_id, prompt_embed, P, W_text, W_cls):
    out = pl.pallas_call(
        _tc_body,
        out_shape=jax.ShapeDtypeStruct((ROWS, 128), jnp.float32),
    )(model_id.astype(jnp.int32).reshape(ROWS, 128),
      prompt_embed.reshape(1, TEXT_DIM), P, W_text, W_cls)
    return out.reshape(BATCH)
